# baseline (speedup 1.0000x reference)
import functools

import jax
import jax.numpy as jnp
from jax.experimental import pallas as pl
from jax.experimental.pallas import tpu as pltpu

B, S, DIM = 8, 576, 256
K = 1024
NQ = 8
N = B * S
COMMIT_W = 0.25

ROW_BLK = 512
N_BLKS = N // ROW_BLK


def _level_body(r_ref, rsq_ref, cb_ref, cbsq_ref, idx_ref, minv_ref):
    r = r_ref[...]
    cb = cb_ref[...]
    rsq = rsq_ref[...].reshape(ROW_BLK, 1)
    cbsq = cbsq_ref[...]
    sc = jax.lax.dot_general(
        r, cb, (((1,), (1,)), ((), ())),
        preferred_element_type=jnp.float32)
    d = (rsq - 2.0 * sc) + cbsq
    minv = jnp.min(d, axis=1)
    iota = jax.lax.broadcasted_iota(jnp.int32, (ROW_BLK, K), 1)
    idx = jnp.min(jnp.where(d == minv[:, None], iota, K), axis=1)
    idx_ref[...] = idx.astype(jnp.int32)
    minv_ref[...] = minv


def _level_argmin(residual, rsq, cb, cbsq):
    return pl.pallas_call(
        _level_body,
        grid=(N_BLKS,),
        in_specs=[
            pl.BlockSpec((ROW_BLK, DIM), lambda i: (i, 0)),
            pl.BlockSpec((ROW_BLK,), lambda i: (i,)),
            pl.BlockSpec((K, DIM), lambda i: (0, 0)),
            pl.BlockSpec((1, K), lambda i: (0, 0)),
        ],
        out_specs=[
            pl.BlockSpec((ROW_BLK,), lambda i: (i,)),
            pl.BlockSpec((ROW_BLK,), lambda i: (i,)),
        ],
        out_shape=[
            jax.ShapeDtypeStruct((N,), jnp.int32),
            jax.ShapeDtypeStruct((N,), jnp.float32),
        ],
    )(residual, rsq, cb, cbsq)


def kernel(x, codebooks):
    x_flat = x.reshape(N, DIM)
    cbsq_all = jnp.sum(codebooks ** 2, axis=-1)
    residual = x_flat
    quantized_sum = jnp.zeros_like(x_flat)
    all_indices = []
    total_commit = jnp.float32(0.0)
    total_perp = jnp.float32(0.0)
    for level in range(NQ):
        cb = codebooks[level]
        rsq = jnp.sum(residual ** 2, axis=-1)
        idx, _minv = _level_argmin(residual, rsq, cb, cbsq_all[level][None, :])
        quantized = jnp.take(cb, idx, axis=0)
        commit = jnp.mean((residual - quantized) ** 2)
        total_commit = total_commit + commit
        counts = jnp.bincount(idx, length=K).astype(jnp.float32)
        avg_probs = counts / N
        perp = jnp.exp(-jnp.sum(avg_probs * jnp.log(avg_probs + 1e-10)))
        total_perp = total_perp + perp
        quantized_sum = quantized_sum + quantized
        residual = residual - quantized
        all_indices.append(idx)
    indices_out = jnp.stack(all_indices, axis=-1).reshape(B, S, NQ)
    q_out = quantized_sum.reshape(B, S, DIM)
    quantized_out = x + jax.lax.stop_gradient(q_out - x)
    return (quantized_out, indices_out, total_commit * COMMIT_W,
            total_perp / NQ)


# speedup vs baseline: 2.7821x; 2.7821x over previous
import jax
import jax.numpy as jnp
from jax.experimental import pallas as pl

B, S, DIM = 8, 576, 256
K = 1024
NQ = 8
N = B * S
COMMIT_W = 0.25

ROW_BLK = 512
N_BLKS = N // ROW_BLK


def _rvq_body(x_ref, cb_ref, cbsq_ref, q_ref, *out_refs):
    idx_refs = out_refs[:NQ]
    counts_ref = out_refs[NQ]
    commit_ref = out_refs[NQ + 1]
    i = pl.program_id(0)

    @pl.when(i == 0)
    def _init():
        counts_ref[...] = jnp.zeros((NQ, K), jnp.float32)
        commit_ref[...] = jnp.zeros((NQ, 128), jnp.float32)

    x = x_ref[...]
    iota = jax.lax.broadcasted_iota(jnp.int32, (ROW_BLK, K), 1)
    r = x
    qsum = jnp.zeros_like(x)
    rsq = jnp.sum(r ** 2, axis=1, keepdims=True)
    for level in range(NQ):
        cb = cb_ref[level * K:(level + 1) * K, :]
        cbsq = cbsq_ref[level:level + 1, :]
        sc = jax.lax.dot_general(
            r, cb, (((1,), (1,)), ((), ())),
            preferred_element_type=jnp.float32)
        d = (rsq - 2.0 * sc) + cbsq
        minv = jnp.min(d, axis=1, keepdims=True)
        idx = jnp.min(jnp.where(d == minv, iota, K), axis=1)
        idx_refs[level][...] = idx.astype(jnp.int32)
        onehot = (iota == idx[:, None]).astype(jnp.float32)
        q = jax.lax.dot_general(
            onehot, cb, (((1,), (0,)), ((), ())),
            preferred_element_type=jnp.float32)
        counts_ref[level:level + 1, :] += jnp.sum(onehot, axis=0,
                                                  keepdims=True)
        qsum = qsum + q
        r = r - q
        rsq = jnp.sum(r ** 2, axis=1, keepdims=True)
        commit_ref[level:level + 1, :] += jnp.sum(rsq)
    q_ref[...] = x + (qsum - x)


def _finalize_body(counts_ref, commit_ref, com_ref, perp_ref):
    counts = counts_ref[...]
    p = counts / N
    ent = jnp.sum(p * jnp.log(p + 1e-10), axis=1, keepdims=True)
    perps = jnp.exp(-ent)
    perp_ref[...] = jnp.full((8, 128), jnp.sum(perps) / NQ)
    commit = jnp.sum(commit_ref[...][:, 0:1]) / (N * DIM)
    com_ref[...] = jnp.full((8, 128), commit * COMMIT_W)


def kernel(x, codebooks):
    x_flat = x.reshape(N, DIM)
    cb_flat = codebooks.reshape(NQ * K, DIM)
    cbsq_all = jnp.sum(codebooks ** 2, axis=-1)

    outs = pl.pallas_call(
        _rvq_body,
        grid=(N_BLKS,),
        in_specs=[
            pl.BlockSpec((ROW_BLK, DIM), lambda i: (i, 0)),
            pl.BlockSpec((NQ * K, DIM), lambda i: (0, 0)),
            pl.BlockSpec((NQ, K), lambda i: (0, 0)),
        ],
        out_specs=[pl.BlockSpec((ROW_BLK, DIM), lambda i: (i, 0))]
        + [pl.BlockSpec((ROW_BLK,), lambda i: (i,)) for _ in range(NQ)]
        + [
            pl.BlockSpec((NQ, K), lambda i: (0, 0)),
            pl.BlockSpec((NQ, 128), lambda i: (0, 0)),
        ],
        out_shape=[jax.ShapeDtypeStruct((N, DIM), jnp.float32)]
        + [jax.ShapeDtypeStruct((N,), jnp.int32) for _ in range(NQ)]
        + [
            jax.ShapeDtypeStruct((NQ, K), jnp.float32),
            jax.ShapeDtypeStruct((NQ, 128), jnp.float32),
        ],
    )(x_flat, cb_flat, cbsq_all)

    quantized = outs[0]
    idx_list = outs[1:1 + NQ]
    counts, commit_acc = outs[1 + NQ], outs[2 + NQ]

    com, perp = pl.pallas_call(
        _finalize_body,
        out_shape=[
            jax.ShapeDtypeStruct((8, 128), jnp.float32),
            jax.ShapeDtypeStruct((8, 128), jnp.float32),
        ],
    )(counts, commit_acc)

    indices_out = jnp.stack(idx_list, axis=-1).reshape(B, S, NQ)
    quantized_out = quantized.reshape(B, S, DIM)
    return (quantized_out, indices_out, com[0, 0], perp[0, 0])
